# baseline (speedup 1.0000x reference)
import jax
import jax.numpy as jnp
from jax import lax
from jax.experimental import pallas as pl
from jax.experimental.pallas import tpu as pltpu

B, NC = 64, 21
NB = 2
GRID = B // NB

_F32 = jnp.float32
_BF16 = jnp.bfloat16


def _dot(a, b):
    return jnp.dot(a, b, preferred_element_type=_F32)


def _dot_ta(a, b):
    return lax.dot_general(a, b, (((0,), (1,)), ((), ())),
                           preferred_element_type=_F32)


def _dot_tb(a, b):
    return lax.dot_general(a, b, (((1,), (1,)), ((), ())),
                           preferred_element_type=_F32)


def _pad_hw(x):
    return jnp.pad(x, ((0, 0), (1, 1), (1, 1), (0, 0)))


def _phases(x):
    n, h2, w2, c = x.shape
    p, q = h2 // 2, w2 // 2
    xr = x.reshape(n, p, 2, w2, c)
    rows = [xr[:, :, 0], xr[:, :, 1]]
    out = []
    for r in rows:
        rr = r.reshape(n, p, q, 2, c)
        out.append([rr[:, :, :, 0, :], rr[:, :, :, 1, :]])
    return out


_S2 = [(0, 0), (1, 0), (0, 1)]


def _conv_s2(xpad, w, b):
    n = xpad.shape[0]
    ph = _phases(xpad)
    ho = xpad.shape[1] // 2 - 1
    taps = []
    for dy in range(3):
        pu, u0 = _S2[dy]
        for dx in range(3):
            pv, v0 = _S2[dx]
            taps.append(ph[pu][pv][:, u0:u0 + ho, v0:v0 + ho, :])
    pat = jnp.concatenate(taps, axis=-1)
    y = _dot(pat.reshape(n * ho * ho, pat.shape[-1]), w) + b
    return y.reshape(n, ho, ho, w.shape[-1])


def _conv_s1(xpad, w, b, hw):
    n = xpad.shape[0]
    taps = []
    cols = [xpad[:, :, dx:dx + hw, :] for dx in range(3)]
    for dy in range(3):
        for dx in range(3):
            taps.append(cols[dx][:, dy:dy + hw])
    pat = jnp.concatenate(taps, axis=-1)
    y = _dot(pat.reshape(n * hw * hw, pat.shape[-1]), w) + b
    return y.reshape(n, hw, hw, w.shape[-1])


def _cmaj_conv(x, w2, bias, hw, k, nb):
    L = hw * hw
    c0 = k // 2
    lanei = lax.broadcasted_iota(jnp.int32, (1, 1, L), 2)
    yi = lanei // hw
    xi = lanei - yi * hw
    xb = x.astype(_BF16)
    pieces = []
    for dy in range(k):
        for dx in range(k):
            off = (dy - c0) * hw + (dx - c0)
            s = off % L
            r = xb if s == 0 else jnp.concatenate([xb[:, :, s:], xb[:, :, :s]],
                                                  axis=2)
            lo_y, hi_y = max(0, c0 - dy), min(hw - 1, hw - 1 + c0 - dy)
            lo_x, hi_x = max(0, c0 - dx), min(hw - 1, hw - 1 + c0 - dx)
            m = ((yi >= lo_y) & (yi <= hi_y) & (xi >= lo_x) & (xi <= hi_x))
            mb = jnp.where(m, 1.0, 0.0).astype(_BF16)
            pieces.append(r * mb)
    xcat = jnp.concatenate(pieces, axis=1)
    outs = []
    for bi in range(nb):
        o = _dot_ta(xcat[bi], w2) + bias
        outs.append(jnp.maximum(o, 0.0).astype(_BF16).reshape(hw, hw, w2.shape[0]))
    return jnp.stack(outs)


def _gn_relu(x, s4, scale, bias, eps=1e-5):
    n, h, w, c = x.shape
    xf = x.astype(_F32)
    cnt = 4.0 * h * w
    sm = jnp.sum(xf, axis=(1, 2))
    sq = jnp.sum(xf * xf, axis=(1, 2))
    gm = _dot(sm, s4) / cnt
    gv = _dot(sq, s4) / cnt - gm * gm
    inv = lax.rsqrt(gv + eps)
    xn = (xf - gm[:, None, None, :]) * inv[:, None, None, :]
    return jnp.maximum(xn * scale + bias, 0.0).astype(_BF16)


def _kernel_a(hid_ref, geo_ref, df_ref, mk_ref,
              g1w_ref, g1b_ref, d1w_ref, d1b_ref, m1w_ref, m1b_ref,
              g2w_ref, g2b_ref, d2w_ref, d2b_ref, m2w_ref, m2b_ref,
              c1whdg_ref, c1wm_ref, c1b_ref, gn1s_ref, gn1b_ref,
              c2w_ref, c2b_ref, gn2s_ref, gn2b_ref,
              c3w_ref, c3b_ref, gn3s_ref, gn3b_ref, s4_ref,
              out_ref):
    relu = lambda v: jnp.maximum(v, 0.0).astype(_BF16)
    s4 = s4_ref[...]

    y1 = _cmaj_conv(geo_ref[...], g1w_ref[...], g1b_ref[...], 64, 3, NB)
    df1 = _cmaj_conv(df_ref[...], d1w_ref[...], d1b_ref[...], 32, 7, NB)
    m1o = _cmaj_conv(mk_ref[...], m1w_ref[...], m1b_ref[...], 32, 3, NB)

    hid = jnp.stack([hid_ref[bi].astype(_BF16).T.reshape(32, 32, 128)
                     for bi in range(NB)])

    geo_enc = relu(_conv_s2(_pad_hw(y1), g2w_ref[...], g2b_ref[...]))
    df_enc = relu(_conv_s1(_pad_hw(df1), d2w_ref[...], d2b_ref[...], 32))
    m_enc = relu(_conv_s1(_pad_hw(m1o), m2w_ref[...], m2b_ref[...], 32))

    encp = _pad_hw(jnp.concatenate([hid, df_enc, geo_enc], axis=-1))
    mp = _pad_hw(m_enc)
    ph_e = _phases(encp)
    ph_m = _phases(mp)
    taps_e, taps_m = [], []
    for dy in range(3):
        pu, u0 = _S2[dy]
        for dx in range(3):
            pv, v0 = _S2[dx]
            taps_e.append(ph_e[pu][pv][:, u0:u0 + 16, v0:v0 + 16, :])
            taps_m.append(ph_m[pu][pv][:, u0:u0 + 16, v0:v0 + 16, :])
    x1 = (_dot(jnp.concatenate(taps_e, axis=-1).reshape(NB * 256, 2304),
               c1whdg_ref[...])
          + _dot(jnp.concatenate(taps_m, axis=-1).reshape(NB * 256, 288),
                 c1wm_ref[...])
          + c1b_ref[...])
    x1 = _gn_relu(x1.reshape(NB, 16, 16, 128), s4, gn1s_ref[...], gn1b_ref[...])

    x2 = _conv_s2(_pad_hw(x1), c2w_ref[...], c2b_ref[...])
    x2 = _gn_relu(x2, s4, gn2s_ref[...], gn2b_ref[...])

    x3 = _conv_s2(_pad_hw(x2), c3w_ref[...], c3b_ref[...])
    x3 = _gn_relu(x3, s4, gn3s_ref[...], gn3b_ref[...])

    x3f = x3.astype(_F32).reshape(NB, 16, 128)
    out_ref[...] = jnp.stack([x3f[bi].T for bi in range(NB)])


def _kernel_b(x_ref, w1_ref, b1_ref, w2_ref, b2_ref,
              rw_ref, rb_ref, tw_ref, tb_ref, mrot_ref, mtr_ref,
              rot_ref, tr_ref):
    relu = lambda v: jnp.maximum(v, 0.0)
    h = relu(_dot_tb(x_ref[...], w1_ref[...]) + b1_ref[...])
    h = relu(_dot_tb(h, w2_ref[...]) + b2_ref[...])
    rot_full = _dot_tb(h, rw_ref[...]) + rb_ref[...]
    tr_full = _dot_tb(h, tw_ref[...]) + tb_ref[...]
    rot_ref[...] = _dot(rot_full, mrot_ref[...])
    tr_ref[...] = _dot(tr_full, mtr_ref[...])


def kernel(hidden_state, delta_flow, mask, geo, obj_id,
           g1_w, g1_b, g2_w, g2_b, d1_w, d1_b, d2_w, d2_b,
           m1_w, m1_b, m2_w, m2_b,
           c1_w, c1_b, gn1_s, gn1_bb, c2_w, c2_b, gn2_s, gn2_bb,
           c3_w, c3_b, gn3_s, gn3_bb,
           fc1_w, fc1_b, fc2_w, fc2_b, rot_w, rot_b, tr_w, tr_b):
    hid_nc = hidden_state.reshape(B, 128, 1024)
    geo_nc = geo.reshape(B, 8, 4096)
    df_nc = jnp.pad(delta_flow, ((0, 0), (0, 6), (0, 0), (0, 0))).reshape(B, 8, 1024)
    mk_nc = jnp.pad(mask, ((0, 0), (0, 7), (0, 0), (0, 0))).reshape(B, 8, 1024)

    g1w2 = g1_w.transpose(0, 2, 3, 1).reshape(128, 72).astype(_BF16)
    d1w2 = jnp.pad(d1_w.transpose(0, 2, 3, 1),
                   ((0, 0), (0, 0), (0, 0), (0, 6))).reshape(128, 392).astype(_BF16)
    m1w2 = jnp.pad(m1_w.transpose(0, 2, 3, 1),
                   ((0, 0), (0, 0), (0, 0), (0, 7))).reshape(64, 72).astype(_BF16)

    def wt(w):
        return w.transpose(2, 3, 1, 0)

    g2w = wt(g2_w).reshape(9 * 128, 64).astype(_BF16)
    d2w = wt(d2_w).reshape(9 * 128, 64).astype(_BF16)
    m2w = wt(m2_w).reshape(9 * 64, 32).astype(_BF16)
    c1t = wt(c1_w)
    c1whdg = jnp.concatenate(
        [c1t[:, :, 0:128], c1t[:, :, 128:192], c1t[:, :, 224:288]],
        axis=2).reshape(9 * 256, 128).astype(_BF16)
    c1wm = c1t[:, :, 192:224].reshape(9 * 32, 128).astype(_BF16)
    c2w = wt(c2_w).reshape(9 * 128, 128).astype(_BF16)
    c3w = wt(c3_w).reshape(9 * 128, 128).astype(_BF16)

    lane = jnp.arange(128)
    s4 = (lane[:, None] // 4 == lane[None, :] // 4).astype(_F32)

    r1 = lambda v: v.reshape(1, -1)

    blocked = (hid_nc, geo_nc, df_nc, mk_nc)
    a_ins = [hid_nc, geo_nc, df_nc, mk_nc,
             g1w2, r1(g1_b), d1w2, r1(d1_b), m1w2, r1(m1_b),
             g2w, r1(g2_b), d2w, r1(d2_b), m2w, r1(m2_b),
             c1whdg, c1wm, r1(c1_b), r1(gn1_s), r1(gn1_bb),
             c2w, r1(c2_b), r1(gn2_s), r1(gn2_bb),
             c3w, r1(c3_b), r1(gn3_s), r1(gn3_bb), s4]

    def bspec(x):
        shp = x.shape
        if any(x is t for t in blocked):
            nz = len(shp) - 1
            return pl.BlockSpec((NB,) + shp[1:], lambda i, _n=nz: (i,) + (0,) * _n)
        nz = len(shp)
        return pl.BlockSpec(shp, lambda i, _n=nz: (0,) * _n)

    aout = pl.pallas_call(
        _kernel_a,
        grid=(GRID,),
        in_specs=[bspec(x) for x in a_ins],
        out_specs=pl.BlockSpec((NB, 128, 16), lambda i: (i, 0, 0)),
        out_shape=jax.ShapeDtypeStruct((B, 128, 16), _F32),
        compiler_params=pltpu.CompilerParams(
            dimension_semantics=("arbitrary",),
            vmem_limit_bytes=52 * 1024 * 1024,
        ),
        name="scflow_convs",
    )(*a_ins)

    xflat = aout.reshape(B, 2048)
    sel = obj_id[0] - 1
    esel = jax.nn.one_hot(sel, NC, dtype=_F32).reshape(NC, 1)
    mrot = jnp.kron(esel, jnp.eye(6, dtype=_F32))
    mtr = jnp.kron(esel, jnp.eye(3, dtype=_F32))

    b_ins = [xflat, fc1_w, r1(fc1_b), fc2_w, r1(fc2_b),
             rot_w, r1(rot_b), tr_w, r1(tr_b), mrot, mtr]

    def bspec_b(x):
        if x is xflat:
            return pl.BlockSpec((32, 2048), lambda i: (i, 0))
        nz = len(x.shape)
        return pl.BlockSpec(x.shape, lambda i, _n=nz: (0,) * _n)

    rot, tr = pl.pallas_call(
        _kernel_b,
        grid=(2,),
        in_specs=[bspec_b(x) for x in b_ins],
        out_specs=[pl.BlockSpec((32, 6), lambda i: (i, 0)),
                   pl.BlockSpec((32, 3), lambda i: (i, 0))],
        out_shape=[jax.ShapeDtypeStruct((B, 6), _F32),
                   jax.ShapeDtypeStruct((B, 3), _F32)],
        compiler_params=pltpu.CompilerParams(
            dimension_semantics=("arbitrary",),
        ),
        name="scflow_fc_heads",
    )(*b_ins)
    return (rot, tr)


# speedup vs baseline: 1.0752x; 1.0752x over previous
import jax
import jax.numpy as jnp
from jax import lax
from jax.experimental import pallas as pl
from jax.experimental.pallas import tpu as pltpu

B, NC = 64, 21
NB = 4
GRID = B // NB

_F32 = jnp.float32


def _dot(a, b):
    return jnp.dot(a, b, preferred_element_type=_F32)


def _dot_ta(a, b):
    return lax.dot_general(a, b, (((0,), (1,)), ((), ())),
                           preferred_element_type=_F32)


def _dot_tb(a, b):
    return lax.dot_general(a, b, (((1,), (1,)), ((), ())),
                           preferred_element_type=_F32)


def _pad_hw(x):
    return jnp.pad(x, ((0, 0), (1, 1), (1, 1), (0, 0)))


def _phases(x):
    n, h2, w2, c = x.shape
    p, q = h2 // 2, w2 // 2
    xr = x.reshape(n, p, 2, w2, c)
    rows = [xr[:, :, 0], xr[:, :, 1]]
    out = []
    for r in rows:
        rr = r.reshape(n, p, q, 2, c)
        out.append([rr[:, :, :, 0, :], rr[:, :, :, 1, :]])
    return out


_S2 = [(0, 0), (1, 0), (0, 1)]


def _conv_s2(xpad, w, b):
    n = xpad.shape[0]
    ph = _phases(xpad)
    ho = xpad.shape[1] // 2 - 1
    taps = []
    for dy in range(3):
        pu, u0 = _S2[dy]
        for dx in range(3):
            pv, v0 = _S2[dx]
            taps.append(ph[pu][pv][:, u0:u0 + ho, v0:v0 + ho, :])
    pat = jnp.concatenate(taps, axis=-1)
    y = _dot(pat.reshape(n * ho * ho, pat.shape[-1]), w) + b
    return y.reshape(n, ho, ho, w.shape[-1])


def _conv_s1(xpad, w, b, hw):
    n = xpad.shape[0]
    taps = []
    cols = [xpad[:, :, dx:dx + hw, :] for dx in range(3)]
    for dy in range(3):
        for dx in range(3):
            taps.append(cols[dx][:, dy:dy + hw])
    pat = jnp.concatenate(taps, axis=-1)
    y = _dot(pat.reshape(n * hw * hw, pat.shape[-1]), w) + b
    return y.reshape(n, hw, hw, w.shape[-1])


def _cmaj_conv(x, w2, bias, hw, k, nb):
    L = hw * hw
    c0 = k // 2
    lanei = lax.broadcasted_iota(jnp.int32, (1, 1, L), 2)
    yi = lanei // hw
    xi = lanei - yi * hw
    pieces = []
    for dy in range(k):
        for dx in range(k):
            off = (dy - c0) * hw + (dx - c0)
            s = off % L
            r = x if s == 0 else jnp.concatenate([x[:, :, s:], x[:, :, :s]],
                                                 axis=2)
            lo_y, hi_y = max(0, c0 - dy), min(hw - 1, hw - 1 + c0 - dy)
            lo_x, hi_x = max(0, c0 - dx), min(hw - 1, hw - 1 + c0 - dx)
            m = ((yi >= lo_y) & (yi <= hi_y) & (xi >= lo_x) & (xi <= hi_x))
            pieces.append(r * jnp.where(m, 1.0, 0.0))
    xcat = jnp.concatenate(pieces, axis=1)
    outs = []
    for bi in range(nb):
        o = _dot_ta(xcat[bi], w2) + bias
        outs.append(jnp.maximum(o, 0.0).reshape(hw, hw, w2.shape[0]))
    return jnp.stack(outs)


def _gn_relu(x, s4, scale, bias, eps=1e-5):
    n, h, w, c = x.shape
    cnt = 4.0 * h * w
    sm = jnp.sum(x, axis=(1, 2))
    sq = jnp.sum(x * x, axis=(1, 2))
    gm = _dot(sm, s4) / cnt
    gv = _dot(sq, s4) / cnt - gm * gm
    inv = lax.rsqrt(gv + eps)
    xn = (x - gm[:, None, None, :]) * inv[:, None, None, :]
    return jnp.maximum(xn * scale + bias, 0.0)


def _kernel_a(hid_ref, geo_ref, df_ref, mk_ref,
              g1w_ref, g1b_ref, d1w_ref, d1b_ref, m1w_ref, m1b_ref,
              g2w_ref, g2b_ref, d2w_ref, d2b_ref, m2w_ref, m2b_ref,
              c1whdg_ref, c1wm_ref, c1b_ref, gn1s_ref, gn1b_ref,
              c2w_ref, c2b_ref, gn2s_ref, gn2b_ref,
              c3w_ref, c3b_ref, gn3s_ref, gn3b_ref, s4_ref,
              out_ref):
    relu = lambda v: jnp.maximum(v, 0.0)
    s4 = s4_ref[...]

    y1 = _cmaj_conv(geo_ref[...], g1w_ref[...], g1b_ref[...], 64, 3, NB)
    df1 = _cmaj_conv(df_ref[...], d1w_ref[...], d1b_ref[...], 32, 7, NB)
    m1o = _cmaj_conv(mk_ref[...], m1w_ref[...], m1b_ref[...], 32, 3, NB)

    hid = jnp.stack([hid_ref[bi].T.reshape(32, 32, 128) for bi in range(NB)])

    geo_enc = relu(_conv_s2(_pad_hw(y1), g2w_ref[...], g2b_ref[...]))
    df_enc = relu(_conv_s1(_pad_hw(df1), d2w_ref[...], d2b_ref[...], 32))
    m_enc = relu(_conv_s1(_pad_hw(m1o), m2w_ref[...], m2b_ref[...], 32))

    encp = _pad_hw(jnp.concatenate([hid, df_enc, geo_enc], axis=-1))
    mp = _pad_hw(m_enc)
    ph_e = _phases(encp)
    ph_m = _phases(mp)
    taps_e, taps_m = [], []
    for dy in range(3):
        pu, u0 = _S2[dy]
        for dx in range(3):
            pv, v0 = _S2[dx]
            taps_e.append(ph_e[pu][pv][:, u0:u0 + 16, v0:v0 + 16, :])
            taps_m.append(ph_m[pu][pv][:, u0:u0 + 16, v0:v0 + 16, :])
    x1 = (_dot(jnp.concatenate(taps_e, axis=-1).reshape(NB * 256, 2304),
               c1whdg_ref[...])
          + _dot(jnp.concatenate(taps_m, axis=-1).reshape(NB * 256, 288),
                 c1wm_ref[...])
          + c1b_ref[...])
    x1 = _gn_relu(x1.reshape(NB, 16, 16, 128), s4, gn1s_ref[...], gn1b_ref[...])

    x2 = _conv_s2(_pad_hw(x1), c2w_ref[...], c2b_ref[...])
    x2 = _gn_relu(x2, s4, gn2s_ref[...], gn2b_ref[...])

    x3 = _conv_s2(_pad_hw(x2), c3w_ref[...], c3b_ref[...])
    x3 = _gn_relu(x3, s4, gn3s_ref[...], gn3b_ref[...])

    x3f = x3.reshape(NB, 16, 128)
    out_ref[...] = jnp.stack([x3f[bi].T for bi in range(NB)])


def _kernel_b(x_ref, w1_ref, b1_ref, w2_ref, b2_ref,
              rw_ref, rb_ref, tw_ref, tb_ref, mrot_ref, mtr_ref,
              rot_ref, tr_ref):
    relu = lambda v: jnp.maximum(v, 0.0)
    h = relu(_dot_tb(x_ref[...], w1_ref[...]) + b1_ref[...])
    h = relu(_dot_tb(h, w2_ref[...]) + b2_ref[...])
    rot_full = _dot_tb(h, rw_ref[...]) + rb_ref[...]
    tr_full = _dot_tb(h, tw_ref[...]) + tb_ref[...]
    rot_ref[...] = _dot(rot_full, mrot_ref[...])
    tr_ref[...] = _dot(tr_full, mtr_ref[...])


def kernel(hidden_state, delta_flow, mask, geo, obj_id,
           g1_w, g1_b, g2_w, g2_b, d1_w, d1_b, d2_w, d2_b,
           m1_w, m1_b, m2_w, m2_b,
           c1_w, c1_b, gn1_s, gn1_bb, c2_w, c2_b, gn2_s, gn2_bb,
           c3_w, c3_b, gn3_s, gn3_bb,
           fc1_w, fc1_b, fc2_w, fc2_b, rot_w, rot_b, tr_w, tr_b):
    hid_nc = hidden_state.reshape(B, 128, 1024)
    geo_nc = geo.reshape(B, 8, 4096)
    df_nc = jnp.pad(delta_flow, ((0, 0), (0, 6), (0, 0), (0, 0))).reshape(B, 8, 1024)
    mk_nc = jnp.pad(mask, ((0, 0), (0, 7), (0, 0), (0, 0))).reshape(B, 8, 1024)

    g1w2 = g1_w.transpose(0, 2, 3, 1).reshape(128, 72)
    d1w2 = jnp.pad(d1_w.transpose(0, 2, 3, 1),
                   ((0, 0), (0, 0), (0, 0), (0, 6))).reshape(128, 392)
    m1w2 = jnp.pad(m1_w.transpose(0, 2, 3, 1),
                   ((0, 0), (0, 0), (0, 0), (0, 7))).reshape(64, 72)

    def wt(w):
        return w.transpose(2, 3, 1, 0)

    g2w = wt(g2_w).reshape(9 * 128, 64)
    d2w = wt(d2_w).reshape(9 * 128, 64)
    m2w = wt(m2_w).reshape(9 * 64, 32)
    c1t = wt(c1_w)
    c1whdg = jnp.concatenate(
        [c1t[:, :, 0:128], c1t[:, :, 128:192], c1t[:, :, 224:288]],
        axis=2).reshape(9 * 256, 128)
    c1wm = c1t[:, :, 192:224].reshape(9 * 32, 128)
    c2w = wt(c2_w).reshape(9 * 128, 128)
    c3w = wt(c3_w).reshape(9 * 128, 128)

    lane = jnp.arange(128)
    s4 = (lane[:, None] // 4 == lane[None, :] // 4).astype(_F32)

    r1 = lambda v: v.reshape(1, -1)

    blocked = (hid_nc, geo_nc, df_nc, mk_nc)
    a_ins = [hid_nc, geo_nc, df_nc, mk_nc,
             g1w2, r1(g1_b), d1w2, r1(d1_b), m1w2, r1(m1_b),
             g2w, r1(g2_b), d2w, r1(d2_b), m2w, r1(m2_b),
             c1whdg, c1wm, r1(c1_b), r1(gn1_s), r1(gn1_bb),
             c2w, r1(c2_b), r1(gn2_s), r1(gn2_bb),
             c3w, r1(c3_b), r1(gn3_s), r1(gn3_bb), s4]

    def bspec(x):
        shp = x.shape
        if any(x is t for t in blocked):
            nz = len(shp) - 1
            return pl.BlockSpec((NB,) + shp[1:], lambda i, _n=nz: (i,) + (0,) * _n)
        nz = len(shp)
        return pl.BlockSpec(shp, lambda i, _n=nz: (0,) * _n)

    aout = pl.pallas_call(
        _kernel_a,
        grid=(GRID,),
        in_specs=[bspec(x) for x in a_ins],
        out_specs=pl.BlockSpec((NB, 128, 16), lambda i: (i, 0, 0)),
        out_shape=jax.ShapeDtypeStruct((B, 128, 16), _F32),
        compiler_params=pltpu.CompilerParams(
            dimension_semantics=("arbitrary",),
            vmem_limit_bytes=52 * 1024 * 1024,
        ),
        name="scflow_convs",
    )(*a_ins)

    xflat = aout.reshape(B, 2048)
    sel = obj_id[0] - 1
    esel = jax.nn.one_hot(sel, NC, dtype=_F32).reshape(NC, 1)
    mrot = jnp.kron(esel, jnp.eye(6, dtype=_F32))
    mtr = jnp.kron(esel, jnp.eye(3, dtype=_F32))

    b_ins = [xflat, fc1_w, r1(fc1_b), fc2_w, r1(fc2_b),
             rot_w, r1(rot_b), tr_w, r1(tr_b), mrot, mtr]

    def bspec_b(x):
        if x is xflat:
            return pl.BlockSpec((32, 2048), lambda i: (i, 0))
        nz = len(x.shape)
        return pl.BlockSpec(x.shape, lambda i, _n=nz: (0,) * _n)

    rot, tr = pl.pallas_call(
        _kernel_b,
        grid=(2,),
        in_specs=[bspec_b(x) for x in b_ins],
        out_specs=[pl.BlockSpec((32, 6), lambda i: (i, 0)),
                   pl.BlockSpec((32, 3), lambda i: (i, 0))],
        out_shape=[jax.ShapeDtypeStruct((B, 6), _F32),
                   jax.ShapeDtypeStruct((B, 3), _F32)],
        compiler_params=pltpu.CompilerParams(
            dimension_semantics=("arbitrary",),
        ),
        name="scflow_fc_heads",
    )(*b_ins)
    return (rot, tr)
